# Initial kernel scaffold; baseline (speedup 1.0000x reference)
import jax
import jax.numpy as jnp
from jax.experimental import pallas as pl


def kernel(x, edge_label_index):
    raise NotImplementedError("write your pallas kernel here")


# speedup vs baseline: 2.2495x; 2.2495x over previous
import functools

import jax
import jax.numpy as jnp
from jax import lax
from jax.experimental import pallas as pl
from jax.experimental.pallas import tpu as pltpu
from jax.experimental.pallas import tpu_sc as plsc

N_NODES = 10000
D_FEAT = 128
HALF = 64
N_EDGES = 320000

NUM_CORES = 2
NUM_SUBCORES = 16
NW = NUM_CORES * NUM_SUBCORES
EDGES_PER_W = N_EDGES // NW
CHUNK = 400
NCHUNKS = EDGES_PER_W // CHUNK
GROUPS = CHUNK // 16


def _make_kernel():
    mesh = plsc.VectorSubcoreMesh(core_axis_name="c", subcore_axis_name="s")

    @functools.partial(
        pl.kernel,
        mesh=mesh,
        compiler_params=pltpu.CompilerParams(
            needs_layout_passes=False, use_tc_tiling_on_sc=False),
        out_type=jax.ShapeDtypeStruct((N_EDGES,), jnp.float32),
        scratch_types=[
            pltpu.VMEM((CHUNK,), jnp.int32),
            pltpu.VMEM((CHUNK,), jnp.int32),
            pltpu.VMEM((CHUNK, HALF), jnp.float32),
            pltpu.VMEM((CHUNK, HALF), jnp.float32),
            pltpu.VMEM((CHUNK,), jnp.float32),
            pltpu.SemaphoreType.DMA,
        ],
    )
    def kern(xs_hbm, xd_hbm, si_hbm, di_hbm, out_hbm,
             si_v, di_v, srows, drows, out_v, sem):
        wid = lax.axis_index("s") * NUM_CORES + lax.axis_index("c")
        base0 = wid * EDGES_PER_W
        lane = lax.iota(jnp.int32, 16)

        def chunk_body(j, carry):
            base = base0 + j * CHUNK
            pltpu.sync_copy(si_hbm.at[pl.ds(base, CHUNK)], si_v)
            pltpu.sync_copy(di_hbm.at[pl.ds(base, CHUNK)], di_v)
            cp1 = pltpu.async_copy(xs_hbm.at[si_v], srows, sem)
            cp2 = pltpu.async_copy(xd_hbm.at[di_v], drows, sem)
            cp1.wait()
            cp2.wait()

            def group_body(g, carry2):
                rows = g * 16 + lane
                acc = jnp.zeros((16,), jnp.float32)
                for d in range(HALF):
                    col = jnp.full((16,), d, jnp.int32)
                    sv = plsc.load_gather(srows, [rows, col])
                    dv = plsc.load_gather(drows, [rows, col])
                    acc = acc + sv * dv
                out_v[pl.ds(g * 16, 16)] = acc
                return carry2

            lax.fori_loop(0, GROUPS, group_body, 0, unroll=False)
            pltpu.sync_copy(out_v, out_hbm.at[pl.ds(base, CHUNK)])
            return carry

        lax.fori_loop(0, NCHUNKS, chunk_body, 0, unroll=False)

    return kern


_KERNEL = _make_kernel()


@jax.jit
def kernel(x, edge_label_index):
    xs = x[:, :HALF]
    xd = x[:, HALF:]
    si = edge_label_index[0]
    di = edge_label_index[1]
    return _KERNEL(xs, xd, si, di)


# speedup vs baseline: 6.6011x; 2.9345x over previous
import functools

import jax
import jax.numpy as jnp
from jax import lax
from jax.experimental import pallas as pl
from jax.experimental.pallas import tpu as pltpu
from jax.experimental.pallas import tpu_sc as plsc

N_NODES = 10000
D_FEAT = 128
HALF = 64
N_EDGES = 320000

NUM_CORES = 2
NUM_SUBCORES = 16
NW = NUM_CORES * NUM_SUBCORES
EDGES_PER_W = N_EDGES // NW
CHUNK = 400
NCHUNKS = EDGES_PER_W // CHUNK
GROUPS = CHUNK // 16


def _make_kernel():
    mesh = plsc.VectorSubcoreMesh(core_axis_name="c", subcore_axis_name="s")

    @functools.partial(
        pl.kernel,
        mesh=mesh,
        compiler_params=pltpu.CompilerParams(
            needs_layout_passes=False, use_tc_tiling_on_sc=False),
        out_type=jax.ShapeDtypeStruct((N_EDGES,), jnp.float32),
        scratch_types=[
            pltpu.VMEM((CHUNK,), jnp.int32),
            pltpu.VMEM((CHUNK,), jnp.int32),
            pltpu.VMEM((CHUNK, HALF), jnp.float32),
            pltpu.VMEM((CHUNK, HALF), jnp.float32),
            pltpu.VMEM((CHUNK,), jnp.float32),
            pltpu.VMEM((256,), jnp.float32),
            pltpu.SemaphoreType.DMA,
        ],
    )
    def kern(xs_hbm, xd_hbm, si_hbm, di_hbm, out_hbm,
             si_v, di_v, srows, drows, out_v, part_v, sem):
        wid = lax.axis_index("s") * NUM_CORES + lax.axis_index("c")
        base0 = wid * EDGES_PER_W
        lane = lax.iota(jnp.int32, 16)

        def chunk_body(j, carry):
            base = base0 + j * CHUNK
            pltpu.sync_copy(si_hbm.at[pl.ds(base, CHUNK)], si_v)
            pltpu.sync_copy(di_hbm.at[pl.ds(base, CHUNK)], di_v)
            cp1 = pltpu.async_copy(xs_hbm.at[si_v], srows, sem)
            cp2 = pltpu.async_copy(xd_hbm.at[di_v], drows, sem)
            cp1.wait()
            cp2.wait()

            def group_body(g, carry2):
                for e in range(16):
                    row = jnp.full((16,), g * 16 + e, jnp.int32)
                    partial = jnp.zeros((16,), jnp.float32)
                    for k in range(HALF // 16):
                        col = k * 16 + lane
                        sv = plsc.load_gather(srows, [row, col])
                        dv = plsc.load_gather(drows, [row, col])
                        partial = partial + sv * dv
                    part_v[pl.ds(e * 16, 16)] = partial
                acc = jnp.zeros((16,), jnp.float32)
                for d in range(16):
                    diag = lane * 16 + ((d + lane) & 15)
                    acc = acc + plsc.load_gather(part_v, [diag])
                out_v[pl.ds(g * 16, 16)] = acc
                return carry2

            lax.fori_loop(0, GROUPS, group_body, 0, unroll=False)
            pltpu.sync_copy(out_v, out_hbm.at[pl.ds(base, CHUNK)])
            return carry

        lax.fori_loop(0, NCHUNKS, chunk_body, 0, unroll=False)

    return kern


_KERNEL = _make_kernel()


@jax.jit
def kernel(x, edge_label_index):
    xs = x[:, :HALF]
    xd = x[:, HALF:]
    si = edge_label_index[0]
    di = edge_label_index[1]
    return _KERNEL(xs, xd, si, di)


# speedup vs baseline: 8.6430x; 1.3093x over previous
import functools

import jax
import jax.numpy as jnp
from jax import lax
from jax.experimental import pallas as pl
from jax.experimental.pallas import tpu as pltpu
from jax.experimental.pallas import tpu_sc as plsc

N_NODES = 10000
D_FEAT = 128
HALF = 64
N_EDGES = 320000

NUM_CORES = 2
NUM_SUBCORES = 16
NW = NUM_CORES * NUM_SUBCORES
EDGES_PER_W = N_EDGES // NW
CHUNK = 400
NCHUNKS = EDGES_PER_W // CHUNK
GROUPS = CHUNK // 16
NPAIRS = NCHUNKS // 2


def _make_kernel():
    mesh = plsc.VectorSubcoreMesh(core_axis_name="c", subcore_axis_name="s")

    @functools.partial(
        pl.kernel,
        mesh=mesh,
        compiler_params=pltpu.CompilerParams(
            needs_layout_passes=False, use_tc_tiling_on_sc=False),
        out_type=jax.ShapeDtypeStruct((N_EDGES,), jnp.float32),
        scratch_types=[
            pltpu.VMEM((CHUNK,), jnp.int32),
            pltpu.VMEM((CHUNK,), jnp.int32),
            pltpu.VMEM((CHUNK, HALF), jnp.float32),
            pltpu.VMEM((CHUNK, HALF), jnp.float32),
            pltpu.VMEM((CHUNK,), jnp.int32),
            pltpu.VMEM((CHUNK,), jnp.int32),
            pltpu.VMEM((CHUNK, HALF), jnp.float32),
            pltpu.VMEM((CHUNK, HALF), jnp.float32),
            pltpu.VMEM((CHUNK,), jnp.float32),
            pltpu.VMEM((256,), jnp.float32),
            pltpu.SemaphoreType.DMA,
            pltpu.SemaphoreType.DMA,
        ],
    )
    def kern(xs_hbm, xd_hbm, si_hbm, di_hbm, out_hbm,
             si_a, di_a, sr_a, dr_a, si_b, di_b, sr_b, dr_b,
             out_v, part_v, sem_a, sem_b):
        wid = lax.axis_index("s") * NUM_CORES + lax.axis_index("c")
        base0 = wid * EDGES_PER_W
        lane = lax.iota(jnp.int32, 16)

        def start(j, si_v, di_v, srows, drows, sem):
            base = base0 + j * CHUNK
            pltpu.sync_copy(si_hbm.at[pl.ds(base, CHUNK)], si_v)
            pltpu.sync_copy(di_hbm.at[pl.ds(base, CHUNK)], di_v)
            pltpu.async_copy(xs_hbm.at[si_v], srows, sem)
            pltpu.async_copy(xd_hbm.at[di_v], drows, sem)

        def drain(si_v, di_v, srows, drows, sem):
            pltpu.make_async_copy(xs_hbm.at[si_v], srows, sem).wait()
            pltpu.make_async_copy(xd_hbm.at[di_v], drows, sem).wait()

        def compute(j, srows, drows):
            def group_body(g, carry2):
                for e in range(16):
                    row = jnp.full((16,), g * 16 + e, jnp.int32)
                    partial = jnp.zeros((16,), jnp.float32)
                    for k in range(HALF // 16):
                        col = k * 16 + lane
                        sv = plsc.load_gather(srows, [row, col])
                        dv = plsc.load_gather(drows, [row, col])
                        partial = partial + sv * dv
                    part_v[pl.ds(e * 16, 16)] = partial
                acc = jnp.zeros((16,), jnp.float32)
                for d in range(16):
                    diag = lane * 16 + ((d + lane) & 15)
                    acc = acc + plsc.load_gather(part_v, [diag])
                out_v[pl.ds(g * 16, 16)] = acc
                return carry2

            lax.fori_loop(0, GROUPS, group_body, 0, unroll=False)
            base = base0 + j * CHUNK
            pltpu.sync_copy(out_v, out_hbm.at[pl.ds(base, CHUNK)])

        start(0, si_a, di_a, sr_a, dr_a, sem_a)

        def pair_body(t, carry):
            j0 = 2 * t
            start(j0 + 1, si_b, di_b, sr_b, dr_b, sem_b)
            drain(si_a, di_a, sr_a, dr_a, sem_a)
            compute(j0, sr_a, dr_a)
            start(j0 + 2, si_a, di_a, sr_a, dr_a, sem_a)
            drain(si_b, di_b, sr_b, dr_b, sem_b)
            compute(j0 + 1, sr_b, dr_b)
            return carry

        lax.fori_loop(0, NPAIRS, pair_body, 0, unroll=False)

        drain(si_a, di_a, sr_a, dr_a, sem_a)
        compute(NCHUNKS - 1, sr_a, dr_a)

    return kern


_KERNEL = _make_kernel()


@jax.jit
def kernel(x, edge_label_index):
    xs = x[:, :HALF]
    xd = x[:, HALF:]
    si = edge_label_index[0]
    di = edge_label_index[1]
    return _KERNEL(xs, xd, si, di)
